# baseline (speedup 1.0000x reference)
import functools

import jax
import jax.numpy as jnp
from jax.experimental import pallas as pl
from jax.experimental.pallas import tpu as pltpu


def _pad_width(H, W):
    Wp = W + 2
    while (H * Wp) % 128:
        Wp += 1
    return Wp


def _starts(Wp):
    return tuple((ky + 1) * Wp + kx - 1 for ky in range(3) for kx in range(3))


def _silu(a):
    return a * jax.nn.sigmoid(a)


def _pack_up_2x(w, cin, cout):
    W = jnp.zeros((4 * cout, 9 * cin), w.dtype)
    for a in range(2):
        for b in range(2):
            r = (2 * a + b) * cout
            for ky in range(3):
                dy = (a + ky - 1) // 2
                for kx in range(3):
                    dx = (b + kx - 1) // 2
                    T = (dy + 1) * 3 + (dx + 1)
                    W = W.at[r:r + cout, T * cin:(T + 1) * cin].add(
                        w[:, (ky * 3 + kx) * cin:(ky * 3 + kx + 1) * cin])
    return W


def _pack_up_2x_on_phases(w, c, cout):
    W = jnp.zeros((16 * cout, 9 * 4 * c), w.dtype)
    for al in range(4):
        for be in range(4):
            r = (4 * al + be) * cout
            for ky in range(3):
                s = (al + ky - 1) // 2
                a = s % 2
                dy = (s - a) // 2
                for kx in range(3):
                    v = (be + kx - 1) // 2
                    b = v % 2
                    dx = (v - b) // 2
                    T = (dy + 1) * 3 + (dx + 1)
                    col = T * (4 * c) + (2 * a + b) * c
                    W = W.at[r:r + cout, col:col + c].add(
                        w[:, (ky * 3 + kx) * c:(ky * 3 + kx + 1) * c])
    return W


def _pack_conv_on_16phases(w, c, cout):
    W = jnp.zeros((16 * cout, 9 * 16 * c), w.dtype)
    for al in range(4):
        for be in range(4):
            r = (4 * al + be) * cout
            for ky in range(3):
                t = al + ky - 1
                a2, dy = t % 4, t // 4
                for kx in range(3):
                    u = be + kx - 1
                    b2, dx = u % 4, u // 4
                    T = (dy + 1) * 3 + (dx + 1)
                    col = T * (16 * c) + (4 * a2 + b2) * c
                    W = W.at[r:r + cout, col:col + c].add(
                        w[:, (ky * 3 + kx) * c:(ky * 3 + kx + 1) * c])
    return W


def _decode_kernel(xf_ref, bmap_ref, s1_ref, s2_ref, wq_ref, wci_ref, bci_ref,
                   w1_ref, ws1_ref, b1_ref, w2_ref, ws2_ref, b2_ref,
                   wo_ref, bo_ref, o_ref, sc0_ref, sc1_ref, sc2_ref,
                   *, H0, W0):
    f32 = jnp.float32
    Wp0 = _pad_width(H0, W0)
    mo0 = H0 * Wp0
    P0 = (H0 + 4) * Wp0
    c0 = wci_ref.shape[0]
    starts = _starts(Wp0)

    lane = jax.lax.broadcasted_iota(jnp.int32, (1, mo0), 1)
    col = lane % Wp0
    colmask = (col >= 1) & (col <= W0)

    def embed(dst_ref, val):
        dst_ref[:, :2 * Wp0] = jnp.zeros_like(dst_ref[:, :2 * Wp0])
        dst_ref[:, 2 * Wp0 + mo0:] = jnp.zeros_like(dst_ref[:, 2 * Wp0 + mo0:])
        dst_ref[:, 2 * Wp0:2 * Wp0 + mo0] = jnp.where(colmask, val, 0.0)

    hq = jnp.dot(wq_ref[...], xf_ref[...], preferred_element_type=f32, precision=jax.lax.Precision.HIGHEST)
    hq = hq + bmap_ref[...]
    patches = jnp.concatenate([hq[:, s:s + mo0] for s in starts], axis=0)
    a = jnp.dot(wci_ref[...], patches, preferred_element_type=f32, precision=jax.lax.Precision.HIGHEST) + bci_ref[...]
    embed(sc0_ref, _silu(a))

    patches = jnp.concatenate([sc0_ref[:, s:s + mo0] for s in starts], axis=0)
    acc = jnp.dot(w1_ref[...], patches, preferred_element_type=f32, precision=jax.lax.Precision.HIGHEST)
    acc = acc + jnp.dot(ws1_ref[...], s1_ref[...], preferred_element_type=f32, precision=jax.lax.Precision.HIGHEST)
    embed(sc1_ref, _silu(acc + b1_ref[...]))

    patches = jnp.concatenate([sc1_ref[:, s:s + mo0] for s in starts], axis=0)
    acc = jnp.dot(w2_ref[...], patches, preferred_element_type=f32, precision=jax.lax.Precision.HIGHEST)
    acc = acc + jnp.dot(ws2_ref[...], s2_ref[...], preferred_element_type=f32, precision=jax.lax.Precision.HIGHEST)
    embed(sc2_ref, _silu(acc + b2_ref[...]))

    cs = sc2_ref.shape[0]
    acc = bo_ref[...] + jnp.zeros((o_ref.shape[0], mo0), f32)
    for t, s in enumerate(starts):
        acc = acc + jnp.dot(wo_ref[:, t * cs:(t + 1) * cs],
                            sc2_ref[:, s:s + mo0], preferred_element_type=f32, precision=jax.lax.Precision.HIGHEST)
    o_ref[...] = jnp.clip(acc, -1.0, 1.0)


def kernel(x, skip1, skip2, pq_w_t, pq_b, conv_in_w, conv_in_b,
           up1_w, skip1_w_t, up1_b, up2_w, skip2_w_t, up2_b,
           conv_out_w, conv_out_b):
    N, _, H0, W0 = x.shape
    cl = pq_w_t.shape[0]
    c0 = conv_in_w.shape[0]
    c1 = up1_w.shape[0]
    cs1 = skip1_w_t.shape[1]
    cs2 = skip2_w_t.shape[1]
    Wp0 = _pad_width(H0, W0)
    P0 = (H0 + 4) * Wp0
    mo0 = H0 * Wp0
    H2, W2 = 4 * H0, 4 * W0
    f32 = jnp.float32

    xf = jnp.pad(x, ((0, 0), (0, cl - x.shape[1]), (2, 2), (1, Wp0 - W0 - 1)))
    xf = xf.reshape(N, cl, P0)

    def phase_split(s, p):
        n, c, _, _ = s.shape
        s = s.reshape(n, c, H0, p, W0, p)
        s = s.transpose(0, 3, 5, 1, 2, 4)
        s = s.reshape(n, p * p * c, H0, W0)
        s = jnp.pad(s, ((0, 0), (0, 0), (0, 0), (1, Wp0 - W0 - 1)))
        return s.reshape(n, p * p * c, mo0)

    s1p = phase_split(skip1, 2)
    s2p = phase_split(skip2, 4)

    rows = jnp.arange(H0 + 4)
    cols = jnp.arange(Wp0)
    valid = ((rows[:, None] >= 2) & (rows[:, None] < H0 + 2)
             & (cols[None, :] >= 1) & (cols[None, :] < W0 + 1)).astype(f32)
    bmap = pq_b[:, None] * valid.reshape(1, P0)

    w1p = _pack_up_2x(up1_w, c0, c1)
    ws1p = jnp.kron(jnp.eye(4, dtype=f32), skip1_w_t)
    b1p = jnp.tile(up1_b, (4, 1))
    w2p = _pack_up_2x_on_phases(up2_w, c1, c1)
    ws2p = jnp.kron(jnp.eye(16, dtype=f32), skip2_w_t)
    b2p = jnp.tile(up2_b, (16, 1))
    wop = _pack_conv_on_16phases(conv_out_w[:3], c1, 3)
    bop = jnp.tile(conv_out_b[:3], (16, 1))

    kern = functools.partial(_decode_kernel, H0=H0, W0=W0)
    bcast = lambda *shape: pl.BlockSpec(shape, lambda n: (0,) * len(shape))
    per_n = lambda *shape: pl.BlockSpec((None,) + shape,
                                        lambda n: (n,) + (0,) * len(shape))
    y = pl.pallas_call(
        kern,
        out_shape=jax.ShapeDtypeStruct((N, 48, mo0), f32),
        grid=(N,),
        in_specs=[
            per_n(cl, P0),
            bcast(cl, P0),
            per_n(4 * cs1, mo0),
            per_n(16 * cs2, mo0),
            bcast(cl, cl),
            bcast(c0, 9 * cl),
            bcast(c0, 1),
            bcast(4 * c1, 9 * c0),
            bcast(4 * c1, 4 * cs1),
            bcast(4 * c1, 1),
            bcast(16 * c1, 36 * c1),
            bcast(16 * c1, 16 * cs2),
            bcast(16 * c1, 1),
            bcast(48, 144 * c1),
            bcast(48, 1),
        ],
        out_specs=per_n(48, mo0),
        scratch_shapes=[
            pltpu.VMEM((c0, P0), f32),
            pltpu.VMEM((4 * c1, P0), f32),
            pltpu.VMEM((16 * c1, P0), f32),
        ],
        compiler_params=pltpu.CompilerParams(
            dimension_semantics=("parallel",),
            vmem_limit_bytes=100 * 1024 * 1024,
        ),
    )(xf, bmap, s1p, s2p, pq_w_t, conv_in_w, conv_in_b,
      w1p, ws1p, b1p, w2p, ws2p, b2p, wop, bop)

    y = y.reshape(N, 4, 4, 3, H0, Wp0)[:, :, :, :, :, 1:W0 + 1]
    y = y.transpose(0, 3, 4, 1, 5, 2)
    return y.reshape(N, 3, H2, W2)


# speedup vs baseline: 1.5050x; 1.5050x over previous
import functools

import jax
import jax.numpy as jnp
from jax.experimental import pallas as pl
from jax.experimental.pallas import tpu as pltpu


def _pad_width(H, W):
    Wp = W + 2
    while (H * Wp) % 128:
        Wp += 1
    return Wp


def _starts(Wp):
    return tuple((ky + 1) * Wp + kx - 1 for ky in range(3) for kx in range(3))


def _silu(a):
    return a * jax.nn.sigmoid(a)


def _pack_up_2x(w, cin, cout):
    Ws = [[jnp.zeros((4 * cout, 9 * cin), w.dtype) for _ in range(2)]
          for _ in range(2)]
    for a in range(2):
        for b in range(2):
            r = (2 * a + b) * cout
            dys = [(a + ky - 1) // 2 for ky in range(3)]
            dxs = [(b + kx - 1) // 2 for kx in range(3)]
            for ky in range(3):
                sy = dys[:ky].count(dys[ky])
                for kx in range(3):
                    sx = dxs[:kx].count(dxs[kx])
                    T = (dys[ky] + 1) * 3 + (dxs[kx] + 1)
                    Ws[sy][sx] = Ws[sy][sx].at[
                        r:r + cout, T * cin:(T + 1) * cin].add(
                        w[:, (ky * 3 + kx) * cin:(ky * 3 + kx + 1) * cin])
    return Ws


def _pack_up_2x_on_phases(w, c, cout):
    Ws = [[jnp.zeros((16 * cout, 9 * 4 * c), w.dtype) for _ in range(2)]
          for _ in range(2)]
    for al in range(4):
        for be in range(4):
            r = (4 * al + be) * cout
            ss = [(al + ky - 1) // 2 for ky in range(3)]
            vs = [(be + kx - 1) // 2 for kx in range(3)]
            for ky in range(3):
                s = ss[ky]
                a, dy = s % 2, (s - s % 2) // 2
                sy = ss[:ky].count(s)
                for kx in range(3):
                    v = vs[kx]
                    b, dx = v % 2, (v - v % 2) // 2
                    sx = vs[:kx].count(v)
                    T = (dy + 1) * 3 + (dx + 1)
                    col = T * (4 * c) + (2 * a + b) * c
                    Ws[sy][sx] = Ws[sy][sx].at[r:r + cout, col:col + c].add(
                        w[:, (ky * 3 + kx) * c:(ky * 3 + kx + 1) * c])
    return Ws


def _pack_conv_on_16phases(w, c, cout):
    W = jnp.zeros((16 * cout, 9 * 16 * c), w.dtype)
    for al in range(4):
        for be in range(4):
            r = (4 * al + be) * cout
            for ky in range(3):
                t = al + ky - 1
                a2, dy = t % 4, t // 4
                for kx in range(3):
                    u = be + kx - 1
                    b2, dx = u % 4, u // 4
                    T = (dy + 1) * 3 + (dx + 1)
                    col = T * (16 * c) + (4 * a2 + b2) * c
                    W = W.at[r:r + cout, col:col + c].add(
                        w[:, (ky * 3 + kx) * c:(ky * 3 + kx + 1) * c])
    return W


def _decode_kernel(xf_ref, bmap_ref, s1_ref, s2_ref, wq_ref, wci_ref, bci_ref,
                   w1_ref, ws1_ref, b1_ref, w2_ref, ws2_ref, b2_ref,
                   wo_ref, bo_ref, o_ref, sc0_ref, sc1_ref, sc2_ref,
                   *, H0, W0):
    f32 = jnp.float32
    Wp0 = _pad_width(H0, W0)
    mo0 = H0 * Wp0
    P0 = (H0 + 4) * Wp0
    c0 = wci_ref.shape[0]
    starts = _starts(Wp0)

    lane = jax.lax.broadcasted_iota(jnp.int32, (1, mo0), 1)
    col = lane % Wp0
    colmask = (col >= 1) & (col <= W0)

    def embed(dst_ref, val):
        dst_ref[:, :2 * Wp0] = jnp.zeros_like(dst_ref[:, :2 * Wp0])
        dst_ref[:, 2 * Wp0 + mo0:] = jnp.zeros_like(dst_ref[:, 2 * Wp0 + mo0:])
        dst_ref[:, 2 * Wp0:2 * Wp0 + mo0] = jnp.where(colmask, val, 0.0)

    hq = jnp.dot(wq_ref[...], xf_ref[...], preferred_element_type=f32)
    hq = hq + bmap_ref[...]
    patches = jnp.concatenate([hq[:, s:s + mo0] for s in starts], axis=0)
    a = jnp.dot(wci_ref[...], patches, preferred_element_type=f32) + bci_ref[...]
    embed(sc0_ref, _silu(a))

    patches = jnp.concatenate([sc0_ref[:, s:s + mo0] for s in starts], axis=0)
    m1 = w1_ref.shape[0] // 4
    y4 = jnp.dot(w1_ref[...], patches, preferred_element_type=f32)
    acc = ((y4[:m1] + y4[m1:2 * m1]) + (y4[2 * m1:3 * m1] + y4[3 * m1:]))
    acc = acc + jnp.dot(ws1_ref[...], s1_ref[...], preferred_element_type=f32)
    embed(sc1_ref, _silu(acc + b1_ref[...]))

    patches = jnp.concatenate([sc1_ref[:, s:s + mo0] for s in starts], axis=0)
    m2 = w2_ref.shape[0] // 4
    y4 = jnp.dot(w2_ref[...], patches, preferred_element_type=f32)
    acc = ((y4[:m2] + y4[m2:2 * m2]) + (y4[2 * m2:3 * m2] + y4[3 * m2:]))
    acc = acc + jnp.dot(ws2_ref[...], s2_ref[...], preferred_element_type=f32)
    embed(sc2_ref, _silu(acc + b2_ref[...]))

    cs = sc2_ref.shape[0]
    acc = bo_ref[...] + jnp.zeros((o_ref.shape[0], mo0), f32)
    for t, s in enumerate(starts):
        acc = acc + jnp.dot(wo_ref[:, t * cs:(t + 1) * cs],
                            sc2_ref[:, s:s + mo0], preferred_element_type=f32)
    o_ref[...] = jnp.clip(acc, -1.0, 1.0)


def kernel(x, skip1, skip2, pq_w_t, pq_b, conv_in_w, conv_in_b,
           up1_w, skip1_w_t, up1_b, up2_w, skip2_w_t, up2_b,
           conv_out_w, conv_out_b):
    N, _, H0, W0 = x.shape
    cl = pq_w_t.shape[0]
    c0 = conv_in_w.shape[0]
    c1 = up1_w.shape[0]
    cs1 = skip1_w_t.shape[1]
    cs2 = skip2_w_t.shape[1]
    Wp0 = _pad_width(H0, W0)
    P0 = (H0 + 4) * Wp0
    mo0 = H0 * Wp0
    H2, W2 = 4 * H0, 4 * W0
    f32 = jnp.float32

    xf = jnp.pad(x, ((0, 0), (0, cl - x.shape[1]), (2, 2), (1, Wp0 - W0 - 1)))
    xf = xf.reshape(N, cl, P0)

    def phase_split(s, p):
        n, c, _, _ = s.shape
        s = s.reshape(n, c, H0, p, W0, p)
        s = s.transpose(0, 3, 5, 1, 2, 4)
        s = s.reshape(n, p * p * c, H0, W0)
        s = jnp.pad(s, ((0, 0), (0, 0), (0, 0), (1, Wp0 - W0 - 1)))
        return s.reshape(n, p * p * c, mo0)

    s1p = phase_split(skip1, 2)
    s2p = phase_split(skip2, 4)

    rows = jnp.arange(H0 + 4)
    cols = jnp.arange(Wp0)
    valid = ((rows[:, None] >= 2) & (rows[:, None] < H0 + 2)
             & (cols[None, :] >= 1) & (cols[None, :] < W0 + 1)).astype(f32)
    bmap = pq_b[:, None] * valid.reshape(1, P0)

    w1s = _pack_up_2x(up1_w, c0, c1)
    w1p = jnp.concatenate([w1s[0][0], w1s[0][1], w1s[1][0], w1s[1][1]],
                          axis=0)
    ws1p = jnp.kron(jnp.eye(4, dtype=f32), skip1_w_t)
    b1p = jnp.tile(up1_b, (4, 1))
    w2s = _pack_up_2x_on_phases(up2_w, c1, c1)
    w2p = jnp.concatenate([w2s[0][0], w2s[0][1], w2s[1][0], w2s[1][1]],
                          axis=0)
    ws2p = jnp.kron(jnp.eye(16, dtype=f32), skip2_w_t)
    b2p = jnp.tile(up2_b, (16, 1))
    wop = _pack_conv_on_16phases(conv_out_w[:3], c1, 3)
    bop = jnp.tile(conv_out_b[:3], (16, 1))

    kern = functools.partial(_decode_kernel, H0=H0, W0=W0)
    bcast = lambda *shape: pl.BlockSpec(shape, lambda n: (0,) * len(shape))
    per_n = lambda *shape: pl.BlockSpec((None,) + shape,
                                        lambda n: (n,) + (0,) * len(shape))
    y = pl.pallas_call(
        kern,
        out_shape=jax.ShapeDtypeStruct((N, 48, mo0), f32),
        grid=(N,),
        in_specs=[
            per_n(cl, P0),
            bcast(cl, P0),
            per_n(4 * cs1, mo0),
            per_n(16 * cs2, mo0),
            bcast(cl, cl),
            bcast(c0, 9 * cl),
            bcast(c0, 1),
            bcast(16 * c1, 9 * c0),
            bcast(4 * c1, 4 * cs1),
            bcast(4 * c1, 1),
            bcast(64 * c1, 36 * c1),
            bcast(16 * c1, 16 * cs2),
            bcast(16 * c1, 1),
            bcast(48, 144 * c1),
            bcast(48, 1),
        ],
        out_specs=per_n(48, mo0),
        scratch_shapes=[
            pltpu.VMEM((c0, P0), f32),
            pltpu.VMEM((4 * c1, P0), f32),
            pltpu.VMEM((16 * c1, P0), f32),
        ],
        compiler_params=pltpu.CompilerParams(
            dimension_semantics=("parallel",),
            vmem_limit_bytes=100 * 1024 * 1024,
        ),
    )(xf, bmap, s1p, s2p, pq_w_t, conv_in_w, conv_in_b,
      w1p, ws1p, b1p, w2p, ws2p, b2p, wop, bop)

    y = y.reshape(N, 4, 4, 3, H0, Wp0)[:, :, :, :, :, 1:W0 + 1]
    y = y.transpose(0, 3, 4, 1, 5, 2)
    return y.reshape(N, 3, H2, W2)


# speedup vs baseline: 1.6651x; 1.1064x over previous
import functools

import jax
import jax.numpy as jnp
from jax.experimental import pallas as pl
from jax.experimental.pallas import tpu as pltpu


def _pad_width(H, W):
    Wp = W + 2
    while (H * Wp) % 128:
        Wp += 1
    return Wp


def _starts(Wp):
    return tuple((ky + 1) * Wp + kx - 1 for ky in range(3) for kx in range(3))


def _silu(a):
    return a * jax.nn.sigmoid(a)


def _pack_up_2x(w, cin, cout):
    Ws = [[jnp.zeros((4 * cout, 9 * cin), w.dtype) for _ in range(2)]
          for _ in range(2)]
    for a in range(2):
        for b in range(2):
            r = (2 * a + b) * cout
            dys = [(a + ky - 1) // 2 for ky in range(3)]
            dxs = [(b + kx - 1) // 2 for kx in range(3)]
            for ky in range(3):
                sy = dys[:ky].count(dys[ky])
                for kx in range(3):
                    sx = dxs[:kx].count(dxs[kx])
                    T = (dys[ky] + 1) * 3 + (dxs[kx] + 1)
                    Ws[sy][sx] = Ws[sy][sx].at[
                        r:r + cout, T * cin:(T + 1) * cin].add(
                        w[:, (ky * 3 + kx) * cin:(ky * 3 + kx + 1) * cin])
    return Ws


def _pack_up_2x_on_phases(w, c, cout):
    Ws = [[jnp.zeros((16 * cout, 9 * 4 * c), w.dtype) for _ in range(2)]
          for _ in range(2)]
    for al in range(4):
        for be in range(4):
            r = (4 * al + be) * cout
            ss = [(al + ky - 1) // 2 for ky in range(3)]
            vs = [(be + kx - 1) // 2 for kx in range(3)]
            for ky in range(3):
                s = ss[ky]
                a, dy = s % 2, (s - s % 2) // 2
                sy = ss[:ky].count(s)
                for kx in range(3):
                    v = vs[kx]
                    b, dx = v % 2, (v - v % 2) // 2
                    sx = vs[:kx].count(v)
                    T = (dy + 1) * 3 + (dx + 1)
                    col = T * (4 * c) + (2 * a + b) * c
                    Ws[sy][sx] = Ws[sy][sx].at[r:r + cout, col:col + c].add(
                        w[:, (ky * 3 + kx) * c:(ky * 3 + kx + 1) * c])
    return Ws


def _pack_conv_on_16phases(w, c, cout):
    W = jnp.zeros((16 * cout, 9 * 16 * c), w.dtype)
    for al in range(4):
        for be in range(4):
            r = (4 * al + be) * cout
            for ky in range(3):
                t = al + ky - 1
                a2, dy = t % 4, t // 4
                for kx in range(3):
                    u = be + kx - 1
                    b2, dx = u % 4, u // 4
                    T = (dy + 1) * 3 + (dx + 1)
                    col = T * (16 * c) + (4 * a2 + b2) * c
                    W = W.at[r:r + cout, col:col + c].add(
                        w[:, (ky * 3 + kx) * c:(ky * 3 + kx + 1) * c])
    return W


def _decode_kernel(xf_ref, bmap_ref, s1_ref, s2_ref, wq_ref, wci_ref, bci_ref,
                   w1_ref, ws1_ref, b1_ref, w2_ref, ws2_ref, b2_ref,
                   wo_ref, bo_ref, o_ref, sc0_ref, sc1_ref, sc2_ref,
                   *, H0, W0):
    f32 = jnp.float32
    bf = jnp.bfloat16
    Wp0 = _pad_width(H0, W0)
    mo0 = H0 * Wp0
    P0 = (H0 + 4) * Wp0
    c0 = wci_ref.shape[0]
    starts = _starts(Wp0)

    lane = jax.lax.broadcasted_iota(jnp.int32, (1, mo0), 1)
    col = lane % Wp0
    colmask = (col >= 1) & (col <= W0)

    def embed(dst_ref, val):
        dst_ref[:, :2 * Wp0] = jnp.zeros_like(dst_ref[:, :2 * Wp0])
        dst_ref[:, 2 * Wp0 + mo0:] = jnp.zeros_like(dst_ref[:, 2 * Wp0 + mo0:])
        dst_ref[:, 2 * Wp0:2 * Wp0 + mo0] = jnp.where(colmask, val, 0.0).astype(dst_ref.dtype)

    hq = jnp.dot(wq_ref[...], xf_ref[...].astype(bf), preferred_element_type=f32)
    hq = (hq + bmap_ref[...]).astype(bf)
    patches = jnp.concatenate([hq[:, s:s + mo0] for s in starts], axis=0)
    a = jnp.dot(wci_ref[...], patches, preferred_element_type=f32) + bci_ref[...]
    embed(sc0_ref, _silu(a))

    patches = jnp.concatenate([sc0_ref[:, s:s + mo0] for s in starts], axis=0)
    m1 = w1_ref.shape[0] // 4
    y4 = jnp.dot(w1_ref[...], patches, preferred_element_type=f32)
    acc = ((y4[:m1] + y4[m1:2 * m1]) + (y4[2 * m1:3 * m1] + y4[3 * m1:]))
    acc = acc + jnp.dot(ws1_ref[...], s1_ref[...], preferred_element_type=f32)
    embed(sc1_ref, _silu(acc + b1_ref[...]))

    patches = jnp.concatenate([sc1_ref[:, s:s + mo0] for s in starts], axis=0)
    m2 = w2_ref.shape[0] // 4
    y4 = jnp.dot(w2_ref[...], patches, preferred_element_type=f32)
    acc = ((y4[:m2] + y4[m2:2 * m2]) + (y4[2 * m2:3 * m2] + y4[3 * m2:]))
    acc = acc + jnp.dot(ws2_ref[...], s2_ref[...], preferred_element_type=f32)
    embed(sc2_ref, _silu(acc + b2_ref[...]))

    cs = sc2_ref.shape[0]
    acc = bo_ref[...] + jnp.zeros((o_ref.shape[0], mo0), f32)
    for t, s in enumerate(starts):
        acc = acc + jnp.dot(wo_ref[:, t * cs:(t + 1) * cs],
                            sc2_ref[:, s:s + mo0], preferred_element_type=f32)
    o_ref[...] = jnp.clip(acc, -1.0, 1.0)


def kernel(x, skip1, skip2, pq_w_t, pq_b, conv_in_w, conv_in_b,
           up1_w, skip1_w_t, up1_b, up2_w, skip2_w_t, up2_b,
           conv_out_w, conv_out_b):
    N, _, H0, W0 = x.shape
    cl = pq_w_t.shape[0]
    c0 = conv_in_w.shape[0]
    c1 = up1_w.shape[0]
    cs1 = skip1_w_t.shape[1]
    cs2 = skip2_w_t.shape[1]
    Wp0 = _pad_width(H0, W0)
    P0 = (H0 + 4) * Wp0
    mo0 = H0 * Wp0
    H2, W2 = 4 * H0, 4 * W0
    f32 = jnp.float32

    xf = jnp.pad(x, ((0, 0), (0, cl - x.shape[1]), (2, 2), (1, Wp0 - W0 - 1)))
    xf = xf.reshape(N, cl, P0)

    def phase_split(s, p):
        n, c, _, _ = s.shape
        s = s.reshape(n, c, H0, p, W0, p)
        s = s.transpose(0, 3, 5, 1, 2, 4)
        s = s.reshape(n, p * p * c, H0, W0)
        s = jnp.pad(s, ((0, 0), (0, 0), (0, 0), (1, Wp0 - W0 - 1)))
        return s.reshape(n, p * p * c, mo0)

    s1p = phase_split(skip1, 2).astype(jnp.bfloat16)
    s2p = phase_split(skip2, 4).astype(jnp.bfloat16)

    rows = jnp.arange(H0 + 4)
    cols = jnp.arange(Wp0)
    valid = ((rows[:, None] >= 2) & (rows[:, None] < H0 + 2)
             & (cols[None, :] >= 1) & (cols[None, :] < W0 + 1)).astype(f32)
    bmap = pq_b[:, None] * valid.reshape(1, P0)

    w1s = _pack_up_2x(up1_w, c0, c1)
    w1p = jnp.concatenate([w1s[0][0], w1s[0][1], w1s[1][0], w1s[1][1]],
                          axis=0)
    ws1p = jnp.kron(jnp.eye(4, dtype=f32), skip1_w_t)
    b1p = jnp.tile(up1_b, (4, 1))
    w2s = _pack_up_2x_on_phases(up2_w, c1, c1)
    w2p = jnp.concatenate([w2s[0][0], w2s[0][1], w2s[1][0], w2s[1][1]],
                          axis=0)
    ws2p = jnp.kron(jnp.eye(16, dtype=f32), skip2_w_t)
    b2p = jnp.tile(up2_b, (16, 1))
    wop = _pack_conv_on_16phases(conv_out_w[:3], c1, 3)
    bop = jnp.tile(conv_out_b[:3], (16, 1))

    kern = functools.partial(_decode_kernel, H0=H0, W0=W0)
    bcast = lambda *shape: pl.BlockSpec(shape, lambda n: (0,) * len(shape))
    per_n = lambda *shape: pl.BlockSpec((None,) + shape,
                                        lambda n: (n,) + (0,) * len(shape))
    bfc = lambda a: a.astype(jnp.bfloat16)
    y = pl.pallas_call(
        kern,
        out_shape=jax.ShapeDtypeStruct((N, 48, mo0), f32),
        grid=(N,),
        in_specs=[
            per_n(cl, P0),
            bcast(cl, P0),
            per_n(4 * cs1, mo0),
            per_n(16 * cs2, mo0),
            bcast(cl, cl),
            bcast(c0, 9 * cl),
            bcast(c0, 1),
            bcast(16 * c1, 9 * c0),
            bcast(4 * c1, 4 * cs1),
            bcast(4 * c1, 1),
            bcast(64 * c1, 36 * c1),
            bcast(16 * c1, 16 * cs2),
            bcast(16 * c1, 1),
            bcast(48, 144 * c1),
            bcast(48, 1),
        ],
        out_specs=per_n(48, mo0),
        scratch_shapes=[
            pltpu.VMEM((c0, P0), jnp.bfloat16),
            pltpu.VMEM((4 * c1, P0), jnp.bfloat16),
            pltpu.VMEM((16 * c1, P0), jnp.bfloat16),
        ],
        compiler_params=pltpu.CompilerParams(
            dimension_semantics=("parallel",),
            vmem_limit_bytes=100 * 1024 * 1024,
        ),
    )(xf, bmap, s1p, s2p, bfc(pq_w_t), bfc(conv_in_w), conv_in_b,
      bfc(w1p), bfc(ws1p), b1p, bfc(w2p), bfc(ws2p), b2p, bfc(wop), bop)

    y = y.reshape(N, 4, 4, 3, H0, Wp0)[:, :, :, :, :, 1:W0 + 1]
    y = y.transpose(0, 3, 4, 1, 5, 2)
    return y.reshape(N, 3, H2, W2)


# speedup vs baseline: 1.8036x; 1.0832x over previous
import functools

import jax
import jax.numpy as jnp
from jax.experimental import pallas as pl
from jax.experimental.pallas import tpu as pltpu


def _pad_width(H, W):
    Wp = W + 2
    while (H * Wp) % 128:
        Wp += 1
    return Wp


def _starts(Wp):
    return tuple((ky + 1) * Wp + kx - 1 for ky in range(3) for kx in range(3))


def _silu(a):
    return a * jax.nn.sigmoid(a)


def _pack_up_2x(w, cin, cout):
    Ws = [[jnp.zeros((4 * cout, 9 * cin), w.dtype) for _ in range(2)]
          for _ in range(2)]
    for a in range(2):
        for b in range(2):
            r = (2 * a + b) * cout
            dys = [(a + ky - 1) // 2 for ky in range(3)]
            dxs = [(b + kx - 1) // 2 for kx in range(3)]
            for ky in range(3):
                sy = dys[:ky].count(dys[ky])
                for kx in range(3):
                    sx = dxs[:kx].count(dxs[kx])
                    T = (dys[ky] + 1) * 3 + (dxs[kx] + 1)
                    Ws[sy][sx] = Ws[sy][sx].at[
                        r:r + cout, T * cin:(T + 1) * cin].add(
                        w[:, (ky * 3 + kx) * cin:(ky * 3 + kx + 1) * cin])
    return Ws


_SV = ((-1, 0), (0, 0), (0, 1), (1, 0), (1, 1), (2, 0))


def _pack_up_2x_on_phases(w, c, cout):
    W = jnp.zeros((16 * cout, 36 * c), w.dtype)
    for al in range(4):
        for be in range(4):
            r = (4 * al + be) * cout
            ss = [(al + ky - 1) // 2 for ky in range(3)]
            vs = [(be + kx - 1) // 2 for kx in range(3)]
            for ky in range(3):
                rv = _SV.index((ss[ky], ss[:ky].count(ss[ky])))
                for kx in range(3):
                    cv = _SV.index((vs[kx], vs[:kx].count(vs[kx])))
                    col = (rv * 6 + cv) * c
                    W = W.at[r:r + cout, col:col + c].add(
                        w[:, (ky * 3 + kx) * c:(ky * 3 + kx + 1) * c])
    return W


def _pack_conv_on_16phases(w, c, cout):
    W = jnp.zeros((16 * cout, 9 * 16 * c), w.dtype)
    for al in range(4):
        for be in range(4):
            r = (4 * al + be) * cout
            for ky in range(3):
                t = al + ky - 1
                a2, dy = t % 4, t // 4
                for kx in range(3):
                    u = be + kx - 1
                    b2, dx = u % 4, u // 4
                    T = (dy + 1) * 3 + (dx + 1)
                    col = T * (16 * c) + (4 * a2 + b2) * c
                    W = W.at[r:r + cout, col:col + c].add(
                        w[:, (ky * 3 + kx) * c:(ky * 3 + kx + 1) * c])
    return W


def _decode_kernel(xf_ref, bmap_ref, s1_ref, s2_ref, wq_ref, wci_ref, bci_ref,
                   w1_ref, ws1_ref, b1_ref, w2_ref, ws2_ref, b2_ref,
                   wo_ref, bo_ref, o_ref, sc0_ref, sc1_ref, sc2_ref,
                   *, H0, W0):
    f32 = jnp.float32
    bf = jnp.bfloat16
    Wp0 = _pad_width(H0, W0)
    mo0 = H0 * Wp0
    P0 = (H0 + 4) * Wp0
    c0 = wci_ref.shape[0]
    starts = _starts(Wp0)

    lane = jax.lax.broadcasted_iota(jnp.int32, (1, mo0), 1)
    col = lane % Wp0
    colmask = (col >= 1) & (col <= W0)

    def embed(dst_ref, val):
        dst_ref[:, :2 * Wp0] = jnp.zeros_like(dst_ref[:, :2 * Wp0])
        dst_ref[:, 2 * Wp0 + mo0:] = jnp.zeros_like(dst_ref[:, 2 * Wp0 + mo0:])
        dst_ref[:, 2 * Wp0:2 * Wp0 + mo0] = jnp.where(colmask, val, 0.0).astype(dst_ref.dtype)

    hq = jnp.dot(wq_ref[...], xf_ref[...].astype(bf), preferred_element_type=f32)
    hq = (hq + bmap_ref[...]).astype(bf)
    patches = jnp.concatenate([hq[:, s:s + mo0] for s in starts], axis=0)
    a = jnp.dot(wci_ref[...], patches, preferred_element_type=f32) + bci_ref[...]
    embed(sc0_ref, _silu(a))

    patches = jnp.concatenate([sc0_ref[:, s:s + mo0] for s in starts], axis=0)
    m1 = w1_ref.shape[0] // 4
    y4 = jnp.dot(w1_ref[...], patches, preferred_element_type=f32)
    acc = ((y4[:m1] + y4[m1:2 * m1]) + (y4[2 * m1:3 * m1] + y4[3 * m1:]))
    acc = acc + jnp.dot(ws1_ref[...], s1_ref[...], preferred_element_type=f32)
    embed(sc1_ref, _silu(acc + b1_ref[...]))

    c1 = sc1_ref.shape[0] // 4
    pieces = []
    for s, _ in _SV:
        a, dy = s % 2, (s - s % 2) // 2
        for v, _ in _SV:
            b, dx = v % 2, (v - v % 2) // 2
            st = (dy + 2) * Wp0 + dx
            rb = (2 * a + b) * c1
            pieces.append(sc1_ref[rb:rb + c1, st:st + mo0])
    patches = jnp.concatenate(pieces, axis=0)
    acc = jnp.dot(w2_ref[...], patches, preferred_element_type=f32)
    acc = acc + jnp.dot(ws2_ref[...], s2_ref[...], preferred_element_type=f32)
    embed(sc2_ref, _silu(acc + b2_ref[...]))

    cs = sc2_ref.shape[0]
    acc = bo_ref[...] + jnp.zeros((o_ref.shape[0], mo0), f32)
    for t, s in enumerate(starts):
        acc = acc + jnp.dot(wo_ref[:, t * cs:(t + 1) * cs],
                            sc2_ref[:, s:s + mo0], preferred_element_type=f32)
    o_ref[...] = jnp.clip(acc, -1.0, 1.0)


def kernel(x, skip1, skip2, pq_w_t, pq_b, conv_in_w, conv_in_b,
           up1_w, skip1_w_t, up1_b, up2_w, skip2_w_t, up2_b,
           conv_out_w, conv_out_b):
    N, _, H0, W0 = x.shape
    cl = pq_w_t.shape[0]
    c0 = conv_in_w.shape[0]
    c1 = up1_w.shape[0]
    cs1 = skip1_w_t.shape[1]
    cs2 = skip2_w_t.shape[1]
    Wp0 = _pad_width(H0, W0)
    P0 = (H0 + 4) * Wp0
    mo0 = H0 * Wp0
    H2, W2 = 4 * H0, 4 * W0
    f32 = jnp.float32

    xf = jnp.pad(x, ((0, 0), (0, cl - x.shape[1]), (2, 2), (1, Wp0 - W0 - 1)))
    xf = xf.reshape(N, cl, P0)

    def phase_split(s, p):
        n, c, _, _ = s.shape
        s = s.reshape(n, c, H0, p, W0, p)
        s = s.transpose(0, 3, 5, 1, 2, 4)
        s = s.reshape(n, p * p * c, H0, W0)
        s = jnp.pad(s, ((0, 0), (0, 0), (0, 0), (1, Wp0 - W0 - 1)))
        return s.reshape(n, p * p * c, mo0)

    s1p = phase_split(skip1, 2).astype(jnp.bfloat16)
    s2p = phase_split(skip2, 4).astype(jnp.bfloat16)

    rows = jnp.arange(H0 + 4)
    cols = jnp.arange(Wp0)
    valid = ((rows[:, None] >= 2) & (rows[:, None] < H0 + 2)
             & (cols[None, :] >= 1) & (cols[None, :] < W0 + 1)).astype(f32)
    bmap = pq_b[:, None] * valid.reshape(1, P0)

    w1s = _pack_up_2x(up1_w, c0, c1)
    w1p = jnp.concatenate([w1s[0][0], w1s[0][1], w1s[1][0], w1s[1][1]],
                          axis=0)
    ws1p = jnp.kron(jnp.eye(4, dtype=f32), skip1_w_t)
    b1p = jnp.tile(up1_b, (4, 1))
    w2p = _pack_up_2x_on_phases(up2_w, c1, c1)
    ws2p = jnp.kron(jnp.eye(16, dtype=f32), skip2_w_t)
    b2p = jnp.tile(up2_b, (16, 1))
    wop = _pack_conv_on_16phases(conv_out_w[:3], c1, 3)
    bop = jnp.tile(conv_out_b[:3], (16, 1))

    kern = functools.partial(_decode_kernel, H0=H0, W0=W0)
    bcast = lambda *shape: pl.BlockSpec(shape, lambda n: (0,) * len(shape))
    per_n = lambda *shape: pl.BlockSpec((None,) + shape,
                                        lambda n: (n,) + (0,) * len(shape))
    bfc = lambda a: a.astype(jnp.bfloat16)
    y = pl.pallas_call(
        kern,
        out_shape=jax.ShapeDtypeStruct((N, 48, mo0), f32),
        grid=(N,),
        in_specs=[
            per_n(cl, P0),
            bcast(cl, P0),
            per_n(4 * cs1, mo0),
            per_n(16 * cs2, mo0),
            bcast(cl, cl),
            bcast(c0, 9 * cl),
            bcast(c0, 1),
            bcast(16 * c1, 9 * c0),
            bcast(4 * c1, 4 * cs1),
            bcast(4 * c1, 1),
            bcast(16 * c1, 36 * c1),
            bcast(16 * c1, 16 * cs2),
            bcast(16 * c1, 1),
            bcast(48, 144 * c1),
            bcast(48, 1),
        ],
        out_specs=per_n(48, mo0),
        scratch_shapes=[
            pltpu.VMEM((c0, P0), jnp.bfloat16),
            pltpu.VMEM((4 * c1, P0), jnp.bfloat16),
            pltpu.VMEM((16 * c1, P0), jnp.bfloat16),
        ],
        compiler_params=pltpu.CompilerParams(
            dimension_semantics=("parallel",),
            vmem_limit_bytes=100 * 1024 * 1024,
        ),
    )(xf, bmap, s1p, s2p, bfc(pq_w_t), bfc(conv_in_w), conv_in_b,
      bfc(w1p), bfc(ws1p), b1p, bfc(w2p), bfc(ws2p), b2p, bfc(wop), bop)

    y = y.reshape(N, 4, 4, 3, H0, Wp0)[:, :, :, :, :, 1:W0 + 1]
    y = y.transpose(0, 3, 4, 1, 5, 2)
    return y.reshape(N, 3, H2, W2)
